# Initial kernel scaffold; baseline (speedup 1.0000x reference)
import functools

import jax
import jax.numpy as jnp
from jax import lax
from jax.experimental import pallas as pl
from jax.experimental.pallas import tpu as pltpu
from jax.experimental.pallas import tpu_sc as plsc

N_NODES = 10000
NP = 10240
D_IN = 128
H_MID = 256
NUM_WORKERS = 32
FPW = 4
LANES = 16


def _pick_chunk(E):
    for ch in (2048, 2000, 1600, 1280, 1024, 800, 640, 512, 400, 320,
               256, 160, 128, 80, 64, 32, 16):
        if E % ch == 0:
            return ch
    raise ValueError(f"edge count {E} not divisible by a usable chunk")


def _sc_aggregate(table_flat, src, dst, F):
    E = src.shape[0]
    CH = _pick_chunk(E)
    P = F // (NUM_WORKERS * FPW)
    assert F % (NUM_WORKERS * FPW) == 0
    mesh = plsc.VectorSubcoreMesh(core_axis_name="c", subcore_axis_name="s")

    def body(table_hbm, src_hbm, dst_hbm, out_hbm, tab_v, acc_v, src_v, dst_v):
        wid = lax.axis_index("s") * 2 + lax.axis_index("c")
        for p in range(P):
            row0 = p * (NUM_WORKERS * FPW) + wid * FPW
            base = row0 * NP
            pltpu.sync_copy(table_hbm.at[pl.ds(base, FPW * NP)], tab_v)

            def zero_body(i, _):
                acc_v[pl.ds(i * LANES, LANES)] = jnp.zeros((LANES,), jnp.float32)
                return 0
            lax.fori_loop(0, (FPW * NP) // LANES, zero_body, 0)

            def chunk_body(k, _):
                pltpu.sync_copy(src_hbm.at[pl.ds(k * CH, CH)], src_v)
                pltpu.sync_copy(dst_hbm.at[pl.ds(k * CH, CH)], dst_v)

                def group_body(g, _):
                    s16 = src_v[pl.ds(g * LANES, LANES)]
                    d16 = dst_v[pl.ds(g * LANES, LANES)]
                    for c in range(FPW):
                        vals = plsc.load_gather(tab_v, [s16 + (c * NP)])
                        plsc.addupdate_scatter(acc_v, [d16 + (c * NP)], vals)
                    return 0
                lax.fori_loop(0, CH // LANES, group_body, 0)
                return 0
            lax.fori_loop(0, E // CH, chunk_body, 0)

            pltpu.sync_copy(acc_v, out_hbm.at[pl.ds(base, FPW * NP)])

    run = pl.kernel(
        body,
        out_type=jax.ShapeDtypeStruct((F * NP,), jnp.float32),
        mesh=mesh,
        scratch_types=[
            pltpu.VMEM((FPW * NP,), jnp.float32),
            pltpu.VMEM((FPW * NP,), jnp.float32),
            pltpu.VMEM((CH,), jnp.int32),
            pltpu.VMEM((CH,), jnp.int32),
        ],
    )
    return run(table_flat, src, dst)


def _tc_layer(aggT, WT, b_col, alpha_col):
    F = aggT.shape[0]
    H = WT.shape[0]
    BN = 2048
    grid = NP // BN

    def body(a_ref, w_ref, b_ref, al_ref, o_ref):
        h = jnp.dot(w_ref[...], a_ref[...], preferred_element_type=jnp.float32)
        h = h + b_ref[...]
        o_ref[...] = jnp.where(h >= 0, h, al_ref[...] * h)

    return pl.pallas_call(
        body,
        grid=(grid,),
        in_specs=[
            pl.BlockSpec((F, BN), lambda i: (0, i)),
            pl.BlockSpec((H, F), lambda i: (0, 0)),
            pl.BlockSpec((H, 1), lambda i: (0, 0)),
            pl.BlockSpec((H, 1), lambda i: (0, 0)),
        ],
        out_specs=pl.BlockSpec((H, BN), lambda i: (0, i)),
        out_shape=jax.ShapeDtypeStruct((H, NP), jnp.float32),
    )(aggT, WT, b_col, alpha_col)


def _tc_head(aggT, WT, b_col, alpha_col, Wd, bd_row):
    F = aggT.shape[0]
    H = WT.shape[0]
    L = Wd.shape[1]
    BN = 2048
    grid = NP // BN

    def body(a_ref, w_ref, b_ref, al_ref, wd_ref, bd_ref, o_ref, pool_ref):
        i = pl.program_id(0)
        h = jnp.dot(w_ref[...], a_ref[...], preferred_element_type=jnp.float32)
        h = h + b_ref[...]
        h = jnp.where(h >= 0, h, al_ref[...] * h)
        col = lax.broadcasted_iota(jnp.int32, (H, BN), 1) + i * BN
        h = jnp.where(col < N_NODES, h, 0.0)
        part = jnp.sum(h, axis=1, keepdims=True)

        @pl.when(i == 0)
        def _():
            pool_ref[...] = part

        @pl.when(i > 0)
        def _():
            pool_ref[...] = pool_ref[...] + part

        @pl.when(i == grid - 1)
        def _():
            pooled = pool_ref[...]
            logits = jnp.sum(pooled * wd_ref[...], axis=0,
                             keepdims=True) + bd_ref[...]
            m = jnp.max(logits, axis=1, keepdims=True)
            e = jnp.exp(logits - m)
            o_ref[...] = e / jnp.sum(e, axis=1, keepdims=True)

    return pl.pallas_call(
        body,
        grid=(grid,),
        in_specs=[
            pl.BlockSpec((F, BN), lambda i: (0, i)),
            pl.BlockSpec((H, F), lambda i: (0, 0)),
            pl.BlockSpec((H, 1), lambda i: (0, 0)),
            pl.BlockSpec((H, 1), lambda i: (0, 0)),
            pl.BlockSpec((H, L), lambda i: (0, 0)),
            pl.BlockSpec((1, L), lambda i: (0, 0)),
        ],
        out_specs=pl.BlockSpec((1, L), lambda i: (0, 0)),
        out_shape=jax.ShapeDtypeStruct((1, L), jnp.float32),
        scratch_shapes=[pltpu.VMEM((H, 1), jnp.float32)],
    )(aggT, WT, b_col, alpha_col, Wd, bd_row)


@jax.jit
def kernel(x, edge_index, W1, b1, alpha1, W2, b2, alpha2, Wd, bd):
    src = edge_index[0]
    dst = edge_index[1]
    xT = jnp.pad(x.T, ((0, 0), (0, NP - N_NODES)))
    agg1 = _sc_aggregate(xT.reshape(-1), src, dst, D_IN)
    h1T = _tc_layer(agg1.reshape(D_IN, NP), W1.T,
                    b1.reshape(-1, 1), alpha1.reshape(-1, 1))
    agg2 = _sc_aggregate(h1T.reshape(-1), src, dst, H_MID)
    out = _tc_head(agg2.reshape(H_MID, NP), W2.T,
                   b2.reshape(-1, 1), alpha2.reshape(-1, 1),
                   Wd, bd.reshape(1, -1))
    return out[0]


# speedup vs baseline: 2.0571x; 2.0571x over previous
import functools

import jax
import jax.numpy as jnp
from jax import lax
from jax.experimental import pallas as pl
from jax.experimental.pallas import tpu as pltpu
from jax.experimental.pallas import tpu_sc as plsc

N_NODES = 10000
NP = 10240
D_IN = 128
H_MID = 256
NUM_WORKERS = 32
FPW = 4
LANES = 16


def _pick_chunk(E):
    for ch in (2048, 2000, 1600, 1280, 1024, 800, 640, 512, 400, 320,
               256, 160, 128, 80, 64, 32, 16):
        if E % ch == 0:
            return ch
    raise ValueError(f"edge count {E} not divisible by a usable chunk")


def _sc_aggregate(table_flat, src, dst, F):
    E = src.shape[0]
    CH = _pick_chunk(E)
    P = F // (NUM_WORKERS * FPW)
    assert F % (NUM_WORKERS * FPW) == 0
    mesh = plsc.VectorSubcoreMesh(core_axis_name="c", subcore_axis_name="s")

    def body(table_hbm, src_hbm, dst_hbm, out_hbm, tab_v, acc_v, src_v, dst_v):
        wid = lax.axis_index("s") * 2 + lax.axis_index("c")
        for p in range(P):
            row0 = p * (NUM_WORKERS * FPW) + wid * FPW
            base = row0 * NP
            pltpu.sync_copy(table_hbm.at[pl.ds(base, FPW * NP)], tab_v)

            def zero_body(i, _):
                acc_v[pl.ds(i * LANES, LANES)] = jnp.zeros((LANES,), jnp.float32)
                return 0
            lax.fori_loop(0, (FPW * NP) // LANES, zero_body, 0)

            def chunk_body(k, _):
                pltpu.sync_copy(src_hbm.at[pl.ds(k * CH, CH)], src_v)
                pltpu.sync_copy(dst_hbm.at[pl.ds(k * CH, CH)], dst_v)

                def group_body(g, _):
                    s16 = src_v[pl.ds(g * LANES, LANES)]
                    d16 = dst_v[pl.ds(g * LANES, LANES)]
                    for c in range(FPW):
                        vals = plsc.load_gather(tab_v, [s16 + (c * NP)])
                        plsc.addupdate_scatter(acc_v, [d16 + (c * NP)], vals)
                    return 0
                lax.fori_loop(0, CH // LANES, group_body, 0)
                return 0
            lax.fori_loop(0, E // CH, chunk_body, 0)

            pltpu.sync_copy(acc_v, out_hbm.at[pl.ds(base, FPW * NP)])

    run = pl.kernel(
        body,
        out_type=jax.ShapeDtypeStruct((F * NP,), jnp.float32),
        mesh=mesh,
        compiler_params=pltpu.CompilerParams(needs_layout_passes=False),
        scratch_types=[
            pltpu.VMEM((FPW * NP,), jnp.float32),
            pltpu.VMEM((FPW * NP,), jnp.float32),
            pltpu.VMEM((CH,), jnp.int32),
            pltpu.VMEM((CH,), jnp.int32),
        ],
    )
    return run(table_flat, src, dst)


def _tc_layer(aggT, WT, b_col, alpha_col):
    F = aggT.shape[0]
    H = WT.shape[0]
    BN = 2048
    grid = NP // BN

    def body(a_ref, w_ref, b_ref, al_ref, o_ref):
        h = jnp.dot(w_ref[...], a_ref[...], preferred_element_type=jnp.float32)
        h = h + b_ref[...]
        o_ref[...] = jnp.where(h >= 0, h, al_ref[...] * h)

    return pl.pallas_call(
        body,
        grid=(grid,),
        in_specs=[
            pl.BlockSpec((F, BN), lambda i: (0, i)),
            pl.BlockSpec((H, F), lambda i: (0, 0)),
            pl.BlockSpec((H, 1), lambda i: (0, 0)),
            pl.BlockSpec((H, 1), lambda i: (0, 0)),
        ],
        out_specs=pl.BlockSpec((H, BN), lambda i: (0, i)),
        out_shape=jax.ShapeDtypeStruct((H, NP), jnp.float32),
    )(aggT, WT, b_col, alpha_col)


def _tc_head(aggT, WT, b_col, alpha_col, Wd, bd_row):
    F = aggT.shape[0]
    H = WT.shape[0]
    L = Wd.shape[1]
    BN = 2048
    grid = NP // BN

    def body(a_ref, w_ref, b_ref, al_ref, wd_ref, bd_ref, o_ref, pool_ref):
        i = pl.program_id(0)
        h = jnp.dot(w_ref[...], a_ref[...], preferred_element_type=jnp.float32)
        h = h + b_ref[...]
        h = jnp.where(h >= 0, h, al_ref[...] * h)
        col = lax.broadcasted_iota(jnp.int32, (H, BN), 1) + i * BN
        h = jnp.where(col < N_NODES, h, 0.0)
        part = jnp.sum(h, axis=1, keepdims=True)

        @pl.when(i == 0)
        def _():
            pool_ref[...] = part

        @pl.when(i > 0)
        def _():
            pool_ref[...] = pool_ref[...] + part

        @pl.when(i == grid - 1)
        def _():
            pooled = pool_ref[...]
            logits = jnp.sum(pooled * wd_ref[...], axis=0,
                             keepdims=True) + bd_ref[...]
            m = jnp.max(logits, axis=1, keepdims=True)
            e = jnp.exp(logits - m)
            o_ref[...] = e / jnp.sum(e, axis=1, keepdims=True)

    return pl.pallas_call(
        body,
        grid=(grid,),
        in_specs=[
            pl.BlockSpec((F, BN), lambda i: (0, i)),
            pl.BlockSpec((H, F), lambda i: (0, 0)),
            pl.BlockSpec((H, 1), lambda i: (0, 0)),
            pl.BlockSpec((H, 1), lambda i: (0, 0)),
            pl.BlockSpec((H, L), lambda i: (0, 0)),
            pl.BlockSpec((1, L), lambda i: (0, 0)),
        ],
        out_specs=pl.BlockSpec((1, L), lambda i: (0, 0)),
        out_shape=jax.ShapeDtypeStruct((1, L), jnp.float32),
        scratch_shapes=[pltpu.VMEM((H, 1), jnp.float32)],
    )(aggT, WT, b_col, alpha_col, Wd, bd_row)


@jax.jit
def kernel(x, edge_index, W1, b1, alpha1, W2, b2, alpha2, Wd, bd):
    src = edge_index[0]
    dst = edge_index[1]
    xT = jnp.pad(x.T, ((0, 0), (0, NP - N_NODES)))
    agg1 = _sc_aggregate(xT.reshape(-1), src, dst, D_IN)
    h1T = _tc_layer(agg1.reshape(D_IN, NP), W1.T,
                    b1.reshape(-1, 1), alpha1.reshape(-1, 1))
    agg2 = _sc_aggregate(h1T.reshape(-1), src, dst, H_MID)
    out = _tc_head(agg2.reshape(H_MID, NP), W2.T,
                   b2.reshape(-1, 1), alpha2.reshape(-1, 1),
                   Wd, bd.reshape(1, -1))
    return out[0]
